# baseline (speedup 1.0000x reference)
import jax
import jax.numpy as jnp
from jax import lax
from jax.experimental import pallas as pl
from jax.experimental.pallas import tpu as pltpu
from jax.experimental.pallas import tpu_sc as plsc

N = 100000
NH = N // 2
D_IN = 144
D_OUT = 128
HID = 64
NUM_SEG = 1024

RBLK = 10000
NBLK = N // RBLK
NBLK_H = NH // RBLK

CHUNK = 128
NCH = N // CHUNK
TAIL = N - NCH * CHUNK
NCH_H = NH // CHUNK
TAIL_H = NH - NCH_H * CHUNK
NW = 32
SEG_PER_TILE = NUM_SEG // 16


def _max_body(z_ref, w1_ref, b1_ref, w2_ref, m_ref, m_acc):
    i = pl.program_id(0)
    h = jnp.maximum(
        jnp.dot(z_ref[...], w1_ref[...], preferred_element_type=jnp.float32)
        + b1_ref[...],
        0.0,
    )
    s = jnp.dot(h, w2_ref[...], preferred_element_type=jnp.float32)
    bm = jnp.max(s)

    @pl.when(i == 0)
    def _():
        m_acc[0] = bm

    @pl.when(i > 0)
    def _():
        m_acc[0] = jnp.maximum(m_acc[0], bm)

    @pl.when(i == NBLK - 1)
    def _():
        m_ref[0, 0] = m_acc[0]


def _scores_max(z, W1, b1, W2):
    return pl.pallas_call(
        _max_body,
        grid=(NBLK,),
        in_specs=[
            pl.BlockSpec((RBLK, D_IN), lambda i: (i, 0)),
            pl.BlockSpec((D_IN, HID), lambda i: (0, 0)),
            pl.BlockSpec((1, HID), lambda i: (0, 0)),
            pl.BlockSpec((HID, 1), lambda i: (0, 0)),
        ],
        out_specs=pl.BlockSpec(memory_space=pltpu.SMEM),
        out_shape=jax.ShapeDtypeStruct((1, 1), jnp.float32),
        scratch_shapes=[pltpu.SMEM((1,), jnp.float32)],
    )(z, W1, b1, W2)


def _vals_body(z_ref, w1_ref, b1_ref, w2_ref, b2_ref, m_ref,
               vals_ref, zsum_ref, z_acc):
    i = pl.program_id(0)
    h = jnp.maximum(
        jnp.dot(z_ref[...], w1_ref[...], preferred_element_type=jnp.float32)
        + b1_ref[...],
        0.0,
    )
    s = jnp.dot(h, w2_ref[...], preferred_element_type=jnp.float32) + b2_ref[0]
    e = jnp.exp(s - m_ref[0])
    vals_ref[...] = e * z_ref[:, :D_OUT].astype(jnp.float32)
    bs = jnp.sum(e)

    @pl.when(i == 0)
    def _():
        z_acc[0] = bs

    @pl.when(i > 0)
    def _():
        z_acc[0] = z_acc[0] + bs

    @pl.when(i == NBLK_H - 1)
    def _():
        zsum_ref[0, 0] = z_acc[0]


def _weighted_vals(z, W1, b1, W2, b2, m, half):
    lo = half * NBLK_H
    return pl.pallas_call(
        _vals_body,
        grid=(NBLK_H,),
        in_specs=[
            pl.BlockSpec((RBLK, D_IN), lambda i: (i + lo, 0)),
            pl.BlockSpec((D_IN, HID), lambda i: (0, 0)),
            pl.BlockSpec((1, HID), lambda i: (0, 0)),
            pl.BlockSpec((HID, 1), lambda i: (0, 0)),
            pl.BlockSpec(memory_space=pltpu.SMEM),
            pl.BlockSpec(memory_space=pltpu.SMEM),
        ],
        out_specs=[
            pl.BlockSpec((RBLK, D_OUT), lambda i: (i, 0)),
            pl.BlockSpec(memory_space=pltpu.SMEM),
        ],
        out_shape=[
            jax.ShapeDtypeStruct((NH, D_OUT), jnp.float32),
            jax.ShapeDtypeStruct((1, 1), jnp.float32),
        ],
        scratch_shapes=[pltpu.SMEM((1,), jnp.float32)],
    )(z, W1, b1, W2, b2, m)


def _sc_cnt_body(idx_hbm, out_cnt, idx_v, idx_tail_v, ones_v, zb_v,
                 acc_cnt, gsem):
    cid = lax.axis_index("c")
    sid = lax.axis_index("s")
    wid = sid * 2 + cid

    zvec = jnp.zeros((16,), jnp.float32)
    onevec = jnp.where(lax.iota(jnp.int32, 16) == 0, 1.0, 0.0).astype(jnp.float32)

    def _zrow(r, _):
        for j in range(D_OUT // 16):
            zb_v[r, pl.ds(j * 16, 16)] = zvec
        return 0

    lax.fori_loop(0, SEG_PER_TILE, _zrow, 0)

    def _orow(r, _):
        ones_v[r, pl.ds(0, 16)] = onevec
        for j in range(1, D_OUT // 16):
            ones_v[r, pl.ds(j * 16, 16)] = zvec
        return 0

    lax.fori_loop(0, CHUNK, _orow, 0)

    pltpu.sync_copy(zb_v, acc_cnt.at[pl.ds(sid * SEG_PER_TILE, SEG_PER_TILE)])
    plsc.subcore_barrier()

    def _start(ch, b):
        pltpu.async_copy(idx_hbm.at[pl.ds(ch * CHUNK, CHUNK)], idx_v.at[b],
                         gsem.at[b])

    def _wait(ch, b):
        pltpu.make_async_copy(idx_hbm.at[pl.ds(ch * CHUNK, CHUNK)],
                              idx_v.at[b], gsem.at[b]).wait()

    _start(wid, 0)

    def _pair(kk, _):
        for b in (0, 1):
            ch = wid + (kk * 2 + b) * NW
            nxt = ch + NW

            @pl.when(ch < NCH)
            def _():
                _wait(ch, b)

                @pl.when(nxt < NCH)
                def _():
                    _start(nxt, 1 - b)

                pltpu.sync_copy(ones_v, acc_cnt.at[idx_v.at[b]], add=True)

        return 0

    nk = N // (CHUNK * NW) + 1
    lax.fori_loop(0, (nk + 1) // 2, _pair, 0)

    @pl.when(wid == NW - 1)
    def _():
        pltpu.sync_copy(idx_hbm.at[pl.ds(NCH * CHUNK, TAIL)], idx_tail_v)
        pltpu.sync_copy(ones_v.at[pl.ds(0, TAIL)],
                        acc_cnt.at[idx_tail_v], add=True)

    plsc.subcore_barrier()
    sl = pl.ds(sid * SEG_PER_TILE, SEG_PER_TILE)
    pltpu.sync_copy(acc_cnt.at[sl], out_cnt.at[cid, sl])


def _sc_counts(idx):
    mesh = plsc.VectorSubcoreMesh(core_axis_name="c", subcore_axis_name="s")
    f = pl.kernel(
        _sc_cnt_body,
        out_type=jax.ShapeDtypeStruct((2, NUM_SEG, D_OUT), jnp.float32),
        mesh=mesh,
        scratch_types=[
            pltpu.VMEM((2, CHUNK), jnp.int32),
            pltpu.VMEM((TAIL,), jnp.int32),
            pltpu.VMEM((CHUNK, D_OUT), jnp.float32),
            pltpu.VMEM((SEG_PER_TILE, D_OUT), jnp.float32),
            pltpu.VMEM_SHARED((NUM_SEG, D_OUT), jnp.float32),
            pltpu.SemaphoreType.DMA((2,)),
        ],
    )
    return f(idx)


def _make_sc_vals_body(lo_row):
    def body(vals_hbm, idx_hbm, out_vals,
             rows_v, idx_v, idx_tail_v, zb_v, acc_vals, gsem):
        cid = lax.axis_index("c")
        sid = lax.axis_index("s")
        wid = sid * 2 + cid

        zvec = jnp.zeros((16,), jnp.float32)

        def _zrow(r, _):
            for j in range(D_OUT // 16):
                zb_v[r, pl.ds(j * 16, 16)] = zvec
            return 0

        lax.fori_loop(0, SEG_PER_TILE, _zrow, 0)

        pltpu.sync_copy(zb_v,
                        acc_vals.at[pl.ds(sid * SEG_PER_TILE, SEG_PER_TILE)])
        plsc.subcore_barrier()

        def _start(ch, b):
            pltpu.async_copy(vals_hbm.at[pl.ds(ch * CHUNK, CHUNK)],
                             rows_v.at[b], gsem.at[b])
            pltpu.async_copy(idx_hbm.at[pl.ds(lo_row + ch * CHUNK, CHUNK)],
                             idx_v.at[b], gsem.at[b])

        def _wait(ch, b):
            pltpu.make_async_copy(vals_hbm.at[pl.ds(ch * CHUNK, CHUNK)],
                                  rows_v.at[b], gsem.at[b]).wait()
            pltpu.make_async_copy(idx_hbm.at[pl.ds(lo_row + ch * CHUNK, CHUNK)],
                                  idx_v.at[b], gsem.at[b]).wait()

        _start(wid, 0)

        def _pair(kk, _):
            for b in (0, 1):
                ch = wid + (kk * 2 + b) * NW
                nxt = ch + NW

                @pl.when(ch < NCH_H)
                def _():
                    _wait(ch, b)

                    @pl.when(nxt < NCH_H)
                    def _():
                        _start(nxt, 1 - b)

                    pltpu.sync_copy(rows_v.at[b], acc_vals.at[idx_v.at[b]],
                                    add=True)

            return 0

        nk = (NCH_H + NW - 1) // NW
        lax.fori_loop(0, (nk + 1) // 2, _pair, 0)

        @pl.when(wid == NW - 1)
        def _():
            base = NCH_H * CHUNK
            pltpu.sync_copy(vals_hbm.at[pl.ds(base, TAIL_H)],
                            rows_v.at[0].at[pl.ds(0, TAIL_H)])
            pltpu.sync_copy(idx_hbm.at[pl.ds(lo_row + base, TAIL_H)],
                            idx_tail_v)
            pltpu.sync_copy(rows_v.at[0].at[pl.ds(0, TAIL_H)],
                            acc_vals.at[idx_tail_v], add=True)

        plsc.subcore_barrier()
        sl = pl.ds(sid * SEG_PER_TILE, SEG_PER_TILE)
        pltpu.sync_copy(acc_vals.at[sl], out_vals.at[cid, sl])

    return body


def _sc_vals(vals_half, idx, half):
    mesh = plsc.VectorSubcoreMesh(core_axis_name="c", subcore_axis_name="s")
    f = pl.kernel(
        _make_sc_vals_body(half * NH),
        out_type=jax.ShapeDtypeStruct((2, NUM_SEG, D_OUT), jnp.float32),
        mesh=mesh,
        scratch_types=[
            pltpu.VMEM((2, CHUNK, D_OUT), jnp.float32),
            pltpu.VMEM((2, CHUNK), jnp.int32),
            pltpu.VMEM((TAIL_H,), jnp.int32),
            pltpu.VMEM((SEG_PER_TILE, D_OUT), jnp.float32),
            pltpu.VMEM_SHARED((NUM_SEG, D_OUT), jnp.float32),
            pltpu.SemaphoreType.DMA((2,)),
        ],
    )
    return f(vals_half, idx)


def _merge_body(a1_ref, a2_ref, c_ref, z1_ref, z2_ref, out_ref):
    v = a1_ref[0] + a1_ref[1] + a2_ref[0] + a2_ref[1]
    c = c_ref[0, :, 0:1] + c_ref[1, :, 0:1]
    zt = z1_ref[0] + z2_ref[0]
    out_ref[...] = v / (zt * jnp.maximum(c, 1.0))


def _merge(a1, a2, cnt, z1, z2):
    return pl.pallas_call(
        _merge_body,
        grid=(1,),
        in_specs=[
            pl.BlockSpec((2, NUM_SEG, D_OUT), lambda i: (0, 0, 0)),
            pl.BlockSpec((2, NUM_SEG, D_OUT), lambda i: (0, 0, 0)),
            pl.BlockSpec((2, NUM_SEG, D_OUT), lambda i: (0, 0, 0)),
            pl.BlockSpec(memory_space=pltpu.SMEM),
            pl.BlockSpec(memory_space=pltpu.SMEM),
        ],
        out_specs=pl.BlockSpec((NUM_SEG, D_OUT), lambda i: (0, 0)),
        out_shape=jax.ShapeDtypeStruct((NUM_SEG, D_OUT), jnp.float32),
    )(a1, a2, cnt, z1, z2)


def kernel(z, batch_index, W1, b1, W2, b2):
    seg = batch_index.astype(jnp.int32)
    z16 = z.astype(jnp.bfloat16)
    W116 = W1.astype(jnp.bfloat16)
    b1r = b1.reshape(1, HID)
    cnt = _sc_counts(seg)
    m = _scores_max(z16, W116, b1r, W2)
    vals1, zs1 = _weighted_vals(z16, W116, b1r, W2, b2.reshape(1),
                                m.reshape(1), 0)
    vals2, zs2 = _weighted_vals(z16, W116, b1r, W2, b2.reshape(1),
                                m.reshape(1), 1)
    a1 = _sc_vals(vals1, seg, 0)
    a2 = _sc_vals(vals2, seg, 1)
    return _merge(a1, a2, cnt, zs1.reshape(1), zs2.reshape(1))


# speedup vs baseline: 1.4637x; 1.4637x over previous
import jax
import jax.numpy as jnp
from jax import lax
from jax.experimental import pallas as pl
from jax.experimental.pallas import tpu as pltpu
from jax.experimental.pallas import tpu_sc as plsc

N = 100000
D_IN = 144
D_OUT = 128
D_HI = D_IN - D_OUT
HID = 64
NUM_SEG = 1024

RBLK = 10000
NBLK = N // RBLK

CHUNK = 128
NCH_FULL = N // CHUNK
TAIL = N - NCH_FULL * CHUNK
NW = 32
KMAX = (NCH_FULL + NW - 1) // NW
SEG_PER_TILE = NUM_SEG // 16


def _max_body(z_ref, w1_ref, b1_ref, w2_ref, m_ref, m_acc):
    i = pl.program_id(0)
    h = jnp.maximum(
        jnp.dot(z_ref[...], w1_ref[...], preferred_element_type=jnp.float32)
        + b1_ref[...],
        0.0,
    )
    s = jnp.dot(h, w2_ref[...], preferred_element_type=jnp.float32)
    bm = jnp.max(s)

    @pl.when(i == 0)
    def _():
        m_acc[0] = bm

    @pl.when(i > 0)
    def _():
        m_acc[0] = jnp.maximum(m_acc[0], bm)

    @pl.when(i == NBLK - 1)
    def _():
        m_ref[0, 0] = m_acc[0]


def _scores_max(z, W1, b1, W2):
    return pl.pallas_call(
        _max_body,
        grid=(NBLK,),
        in_specs=[
            pl.BlockSpec((RBLK, D_IN), lambda i: (i, 0)),
            pl.BlockSpec((D_IN, HID), lambda i: (0, 0)),
            pl.BlockSpec((1, HID), lambda i: (0, 0)),
            pl.BlockSpec((HID, 1), lambda i: (0, 0)),
        ],
        out_specs=pl.BlockSpec(memory_space=pltpu.SMEM),
        out_shape=jax.ShapeDtypeStruct((1, 1), jnp.float32),
        scratch_shapes=[pltpu.SMEM((1,), jnp.float32)],
    )(z, W1, b1, W2)


def _vals_body(z_ref, w1_ref, b1_ref, w2_ref, b2_ref, m_ref,
               vals_ref, zsum_ref, z_acc):
    i = pl.program_id(0)
    h = jnp.maximum(
        jnp.dot(z_ref[...], w1_ref[...], preferred_element_type=jnp.float32)
        + b1_ref[...],
        0.0,
    )
    s = jnp.dot(h, w2_ref[...], preferred_element_type=jnp.float32) + b2_ref[0]
    e = jnp.exp(s - m_ref[0])
    vals_ref[...] = e * z_ref[:, :D_OUT].astype(jnp.float32)
    bs = jnp.sum(e)

    @pl.when(i == 0)
    def _():
        z_acc[0] = bs

    @pl.when(i > 0)
    def _():
        z_acc[0] = z_acc[0] + bs

    @pl.when(i == NBLK - 1)
    def _():
        zsum_ref[0, 0] = z_acc[0]


def _weighted_vals(z, W1, b1, W2, b2, m):
    return pl.pallas_call(
        _vals_body,
        grid=(NBLK,),
        in_specs=[
            pl.BlockSpec((RBLK, D_IN), lambda i: (i, 0)),
            pl.BlockSpec((D_IN, HID), lambda i: (0, 0)),
            pl.BlockSpec((1, HID), lambda i: (0, 0)),
            pl.BlockSpec((HID, 1), lambda i: (0, 0)),
            pl.BlockSpec(memory_space=pltpu.SMEM),
            pl.BlockSpec(memory_space=pltpu.SMEM),
        ],
        out_specs=[
            pl.BlockSpec((RBLK, D_OUT), lambda i: (i, 0)),
            pl.BlockSpec(memory_space=pltpu.SMEM),
        ],
        out_shape=[
            jax.ShapeDtypeStruct((N, D_OUT), jnp.float32),
            jax.ShapeDtypeStruct((1, 1), jnp.float32),
        ],
        scratch_shapes=[pltpu.SMEM((1,), jnp.float32)],
    )(z, W1, b1, W2, b2, m)


def _sc_pool_body(vals_hbm, idx_hbm, out_vals, out_cnt,
                  rows_v, idx_v, idx_tail_v, ones_v, zb_v,
                  acc_vals, acc_cnt, gsem, csem):
    cid = lax.axis_index("c")
    sid = lax.axis_index("s")
    wid = sid * 2 + cid

    zvec = jnp.zeros((16,), jnp.float32)
    onevec = jnp.where(lax.iota(jnp.int32, 16) == 0, 1.0, 0.0).astype(jnp.float32)

    def _zrow(r, _):
        for j in range(D_OUT // 16):
            zb_v[r, pl.ds(j * 16, 16)] = zvec
        return 0

    lax.fori_loop(0, SEG_PER_TILE, _zrow, 0)

    def _orow(r, _):
        ones_v[r, pl.ds(0, 16)] = onevec
        for j in range(1, D_OUT // 16):
            ones_v[r, pl.ds(j * 16, 16)] = zvec
        return 0

    lax.fori_loop(0, CHUNK, _orow, 0)

    pltpu.sync_copy(zb_v, acc_vals.at[pl.ds(sid * SEG_PER_TILE, SEG_PER_TILE)])
    pltpu.sync_copy(zb_v, acc_cnt.at[pl.ds(sid * SEG_PER_TILE, SEG_PER_TILE)])
    plsc.subcore_barrier()

    def _start_gather(ch, b):
        base = ch * CHUNK
        pltpu.async_copy(vals_hbm.at[pl.ds(base, CHUNK)], rows_v.at[b], gsem.at[b])
        pltpu.async_copy(idx_hbm.at[pl.ds(base, CHUNK)], idx_v.at[b], gsem.at[b])

    def _wait_gather(ch, b):
        base = ch * CHUNK
        pltpu.make_async_copy(
            vals_hbm.at[pl.ds(base, CHUNK)], rows_v.at[b], gsem.at[b]).wait()
        pltpu.make_async_copy(
            idx_hbm.at[pl.ds(base, CHUNK)], idx_v.at[b], gsem.at[b]).wait()

    _start_gather(wid, 0)

    def _pair(kk, _):
        k2 = kk * 2
        for b in (0, 1):
            k = k2 + b
            ch = wid + k * NW
            nxt = ch + NW

            @pl.when(ch < NCH_FULL)
            def _():
                _wait_gather(ch, b)

                @pl.when(nxt < NCH_FULL)
                def _():
                    _start_gather(nxt, 1 - b)

                pltpu.async_copy(ones_v, acc_cnt.at[idx_v.at[b]], csem, add=True)
                pltpu.sync_copy(rows_v.at[b], acc_vals.at[idx_v.at[b]], add=True)
                pltpu.make_async_copy(
                    ones_v, acc_cnt.at[idx_v.at[b]], csem).wait()

        return 0

    lax.fori_loop(0, (KMAX + 1) // 2, _pair, 0)

    @pl.when(wid == NW - 1)
    def _():
        base = NCH_FULL * CHUNK
        pltpu.sync_copy(vals_hbm.at[pl.ds(base, TAIL)],
                        rows_v.at[0].at[pl.ds(0, TAIL)])
        pltpu.sync_copy(idx_hbm.at[pl.ds(base, TAIL)], idx_tail_v)
        pltpu.sync_copy(rows_v.at[0].at[pl.ds(0, TAIL)],
                        acc_vals.at[idx_tail_v], add=True)
        pltpu.sync_copy(ones_v.at[pl.ds(0, TAIL)],
                        acc_cnt.at[idx_tail_v], add=True)

    plsc.subcore_barrier()

    sl = pl.ds(sid * SEG_PER_TILE, SEG_PER_TILE)
    pltpu.sync_copy(acc_vals.at[sl], out_vals.at[cid, sl])
    pltpu.sync_copy(acc_cnt.at[sl], out_cnt.at[cid, sl])


def _sc_pool(vals, idx):
    mesh = plsc.VectorSubcoreMesh(core_axis_name="c", subcore_axis_name="s")
    f = pl.kernel(
        _sc_pool_body,
        out_type=(
            jax.ShapeDtypeStruct((2, NUM_SEG, D_OUT), jnp.float32),
            jax.ShapeDtypeStruct((2, NUM_SEG, D_OUT), jnp.float32),
        ),
        mesh=mesh,
        scratch_types=[
            pltpu.VMEM((2, CHUNK, D_OUT), jnp.float32),
            pltpu.VMEM((2, CHUNK), jnp.int32),
            pltpu.VMEM((TAIL,), jnp.int32),
            pltpu.VMEM((CHUNK, D_OUT), jnp.float32),
            pltpu.VMEM((SEG_PER_TILE, D_OUT), jnp.float32),
            pltpu.VMEM_SHARED((NUM_SEG, D_OUT), jnp.float32),
            pltpu.VMEM_SHARED((NUM_SEG, D_OUT), jnp.float32),
            pltpu.SemaphoreType.DMA((2,)),
            pltpu.SemaphoreType.DMA,
        ],
    )
    return f(vals, idx)


def _merge_body(av_ref, ac_ref, zsum_ref, out_ref):
    v = av_ref[0] + av_ref[1]
    c = ac_ref[0, :, 0:1] + ac_ref[1, :, 0:1]
    zt = zsum_ref[0]
    out_ref[...] = v / (zt * jnp.maximum(c, 1.0))


def _merge(acc_vals, acc_cnt, zsum):
    return pl.pallas_call(
        _merge_body,
        grid=(1,),
        in_specs=[
            pl.BlockSpec((2, NUM_SEG, D_OUT), lambda i: (0, 0, 0)),
            pl.BlockSpec((2, NUM_SEG, D_OUT), lambda i: (0, 0, 0)),
            pl.BlockSpec(memory_space=pltpu.SMEM),
        ],
        out_specs=pl.BlockSpec((NUM_SEG, D_OUT), lambda i: (0, 0)),
        out_shape=jax.ShapeDtypeStruct((NUM_SEG, D_OUT), jnp.float32),
    )(acc_vals, acc_cnt, zsum)


def kernel(z, batch_index, W1, b1, W2, b2):
    seg = batch_index.astype(jnp.int32)
    z16 = z.astype(jnp.bfloat16)
    W116 = W1.astype(jnp.bfloat16)
    b1r = b1.reshape(1, HID)
    m = _scores_max(z16, W116, b1r, W2)
    vals, zsum = _weighted_vals(z16, W116, b1r, W2, b2.reshape(1), m.reshape(1))
    acc_vals, acc_cnt = _sc_pool(vals, seg)
    return _merge(acc_vals, acc_cnt, zsum.reshape(1))
